# baseline (speedup 1.0000x reference)
import jax
import jax.numpy as jnp
from jax.experimental import pallas as pl
from jax.experimental.pallas import tpu as pltpu

N = 100000
C = 256
W = 1.0
BR = 10000
GD = N // BR


def _fused_body(l_ref, p_ref, s_ref, t_ref, loss_ref, pre_ref, post_ref):
    i = pl.program_id(0)
    labr = l_ref[0]
    p = p_ref[...]
    s = s_ref[...]
    t = t_ref[...]

    pos = (labr >= 0) & (labr < C)
    labm = jnp.where(pos, labr, -1)
    part_pre = jnp.sum(jnp.where(pos, 1.0, 0.0))

    labc = labm.reshape(BR, 1)
    col = jax.lax.broadcasted_iota(jnp.int32, (BR, C), 1)
    sel = col == labc

    sp = jnp.log1p(jnp.exp(p))
    d = s - t
    dd = jnp.where(sel, -d, d)
    m = jnp.maximum(dd, 0.0)
    loss = (sp - jnp.where(sel, p, 0.0)) * m * m
    postv = jnp.where(sel & (dd > 0), 1.0, 0.0)

    ones_row = jnp.ones((1, BR), jnp.float32)
    dims = (((1,), (0,)), ((), ()))
    loss_c = jax.lax.dot_general(ones_row, loss, dims,
                                 preferred_element_type=jnp.float32)
    post_c = jax.lax.dot_general(ones_row, postv, dims,
                                 preferred_element_type=jnp.float32)
    part_loss = jnp.sum(loss_c)
    part_post = jnp.sum(post_c)

    @pl.when(i == 0)
    def _():
        loss_ref[0, 0] = 0.0
        pre_ref[0, 0] = 0.0
        post_ref[0, 0] = 0.0

    loss_ref[0, 0] += part_loss
    pre_ref[0, 0] += part_pre
    post_ref[0, 0] += part_post


_fused = pl.pallas_call(
    _fused_body,
    grid=(GD,),
    in_specs=[pl.BlockSpec((1, 1, BR), lambda i: (i, 0, 0))] + [
        pl.BlockSpec((BR, C), lambda i: (i, 0))] * 3,
    out_specs=[pl.BlockSpec(memory_space=pltpu.SMEM)] * 3,
    out_shape=[jax.ShapeDtypeStruct((1, 1), jnp.float32)] * 3,
    compiler_params=pltpu.CompilerParams(vmem_limit_bytes=100 * 1024 * 1024),
)


def kernel(pred, label, stu_score, tea_score):
    lab2d = label.astype(jnp.int32).reshape(GD, 1, BR)
    loss, pre, post = _fused(lab2d, pred, stu_score, tea_score)
    loss_cls = loss[0, 0] * (W / N)
    return (loss_cls, pre[0, 0], post[0, 0])
